# baseline (device time: 45832 ns/iter reference)
import jax
import jax.numpy as jnp
from jax import lax
from jax.experimental import pallas as pl
from jax.experimental.pallas import tpu as pltpu

N_DEV = 4


def kernel(x, w_mat):
    m_per, k = x.shape
    _, n_shard = w_mat.shape

    def body(x_ref, w_ref, out_ref, comm_ref, send_sems, recv_sems):
        my_pos = lax.axis_index("i")
        left = (my_pos - 1) % N_DEV
        right = (my_pos + 1) % N_DEV

        barrier_sem = pltpu.get_barrier_semaphore()
        for nbr in [left, right]:
            pl.semaphore_signal(
                barrier_sem, inc=1,
                device_id=(nbr,), device_id_type=pl.DeviceIdType.MESH,
            )
        pl.semaphore_wait(barrier_sem, 2)

        for h in range(N_DEV - 1):
            src = x_ref if h == 0 else comm_ref.at[h - 1]
            rdma = pltpu.make_async_remote_copy(
                src_ref=src,
                dst_ref=comm_ref.at[h],
                send_sem=send_sems.at[h],
                recv_sem=recv_sems.at[h],
                device_id=(right,),
                device_id_type=pl.DeviceIdType.MESH,
            )
            rdma.start()

            origin = (my_pos - h) % N_DEV
            chunk = x_ref[:, :] if h == 0 else comm_ref[h - 1, :, :]
            out_ref[pl.ds(origin * m_per, m_per), :] = jnp.dot(
                chunk, w_ref[:, :], preferred_element_type=jnp.float32
            )

            rdma.wait_recv()
            rdma.wait_send()

        origin = (my_pos - (N_DEV - 1)) % N_DEV
        out_ref[pl.ds(origin * m_per, m_per), :] = jnp.dot(
            comm_ref[N_DEV - 2, :, :], w_ref[:, :],
            preferred_element_type=jnp.float32,
        )

    out_shape = jax.ShapeDtypeStruct((N_DEV * m_per, n_shard), jnp.float32)
    return pl.pallas_call(
        body,
        out_shape=out_shape,
        in_specs=[
            pl.BlockSpec(memory_space=pltpu.VMEM),
            pl.BlockSpec(memory_space=pltpu.VMEM),
        ],
        out_specs=pl.BlockSpec(memory_space=pltpu.VMEM),
        scratch_shapes=[
            pltpu.VMEM((N_DEV - 1, m_per, k), jnp.float32),
            pltpu.SemaphoreType.DMA((N_DEV - 1,)),
            pltpu.SemaphoreType.DMA((N_DEV - 1,)),
        ],
        compiler_params=pltpu.CompilerParams(collective_id=0),
    )(x, w_mat)


# device time: 27052 ns/iter; 1.6942x vs baseline; 1.6942x over previous
import jax
import jax.numpy as jnp
from jax import lax
from jax.experimental import pallas as pl
from jax.experimental.pallas import tpu as pltpu

N_DEV = 4


def kernel(x, w_mat):
    m_per, k = x.shape
    _, n_shard = w_mat.shape
    half = m_per // 2

    def body(x_ref, w_ref, out_ref, cw0, ccw0, cw1, ccw1,
             send_sems, recv_sems):
        my_pos = lax.axis_index("i")
        left = (my_pos - 1) % N_DEV
        right = (my_pos + 1) % N_DEV

        barrier_sem = pltpu.get_barrier_semaphore()
        for nbr in [left, right]:
            pl.semaphore_signal(
                barrier_sem, inc=1,
                device_id=(nbr,), device_id_type=pl.DeviceIdType.MESH,
            )
        pl.semaphore_wait(barrier_sem, 2)

        h0_cw = pltpu.make_async_remote_copy(
            src_ref=x_ref, dst_ref=cw0,
            send_sem=send_sems.at[0], recv_sem=recv_sems.at[0],
            device_id=(right,), device_id_type=pl.DeviceIdType.MESH,
        )
        h0_ccw = pltpu.make_async_remote_copy(
            src_ref=x_ref, dst_ref=ccw0,
            send_sem=send_sems.at[1], recv_sem=recv_sems.at[1],
            device_id=(left,), device_id_type=pl.DeviceIdType.MESH,
        )
        h0_cw.start()
        h0_ccw.start()

        out_ref[pl.ds(my_pos * m_per, m_per), :] = jnp.dot(
            x_ref[:, :], w_ref[:, :], preferred_element_type=jnp.float32
        )

        h0_cw.wait_recv()
        h0_ccw.wait_recv()

        h1_cw = pltpu.make_async_remote_copy(
            src_ref=cw0.at[pl.ds(0, half)], dst_ref=cw1,
            send_sem=send_sems.at[2], recv_sem=recv_sems.at[2],
            device_id=(right,), device_id_type=pl.DeviceIdType.MESH,
        )
        h1_ccw = pltpu.make_async_remote_copy(
            src_ref=ccw0.at[pl.ds(half, half)], dst_ref=ccw1,
            send_sem=send_sems.at[3], recv_sem=recv_sems.at[3],
            device_id=(left,), device_id_type=pl.DeviceIdType.MESH,
        )
        h1_cw.start()
        h1_ccw.start()

        out_ref[pl.ds(left * m_per, m_per), :] = jnp.dot(
            cw0[:, :], w_ref[:, :], preferred_element_type=jnp.float32
        )
        out_ref[pl.ds(right * m_per, m_per), :] = jnp.dot(
            ccw0[:, :], w_ref[:, :], preferred_element_type=jnp.float32
        )

        h1_cw.wait_recv()
        h1_ccw.wait_recv()

        diag = (my_pos + 2) % N_DEV
        out_ref[pl.ds(diag * m_per, half), :] = jnp.dot(
            cw1[:, :], w_ref[:, :], preferred_element_type=jnp.float32
        )
        out_ref[pl.ds(diag * m_per + half, half), :] = jnp.dot(
            ccw1[:, :], w_ref[:, :], preferred_element_type=jnp.float32
        )

        h0_cw.wait_send()
        h0_ccw.wait_send()
        h1_cw.wait_send()
        h1_ccw.wait_send()

    out_shape = jax.ShapeDtypeStruct((N_DEV * m_per, n_shard), jnp.float32)
    return pl.pallas_call(
        body,
        out_shape=out_shape,
        in_specs=[
            pl.BlockSpec(memory_space=pltpu.VMEM),
            pl.BlockSpec(memory_space=pltpu.VMEM),
        ],
        out_specs=pl.BlockSpec(memory_space=pltpu.VMEM),
        scratch_shapes=[
            pltpu.VMEM((m_per, k), jnp.float32),
            pltpu.VMEM((m_per, k), jnp.float32),
            pltpu.VMEM((half, k), jnp.float32),
            pltpu.VMEM((half, k), jnp.float32),
            pltpu.SemaphoreType.DMA((4,)),
            pltpu.SemaphoreType.DMA((4,)),
        ],
        compiler_params=pltpu.CompilerParams(collective_id=0),
    )(x, w_mat)


# device time: 25939 ns/iter; 1.7669x vs baseline; 1.0429x over previous
import jax
import jax.numpy as jnp
from jax import lax
from jax.experimental import pallas as pl
from jax.experimental.pallas import tpu as pltpu

N_DEV = 4


def kernel(x, w_mat):
    m_per, k = x.shape
    _, n_shard = w_mat.shape
    half = m_per // 2

    def body(x_ref, w_ref, out_ref, cw0, ccw0, cw1, ccw1,
             send_sems, recv_sems):
        my_pos = lax.axis_index("i")
        left = (my_pos - 1) % N_DEV
        right = (my_pos + 1) % N_DEV

        barrier_sem = pltpu.get_barrier_semaphore()
        for nbr in [left, right]:
            pl.semaphore_signal(
                barrier_sem, inc=1,
                device_id=(nbr,), device_id_type=pl.DeviceIdType.MESH,
            )
        pl.semaphore_wait(barrier_sem, 2)

        def remote_copy(src, dst, sem_idx, dev):
            return pltpu.make_async_remote_copy(
                src_ref=src, dst_ref=dst,
                send_sem=send_sems.at[sem_idx], recv_sem=recv_sems.at[sem_idx],
                device_id=(dev,), device_id_type=pl.DeviceIdType.MESH,
            )

        top = pl.ds(0, half)
        bot = pl.ds(half, half)

        cw_a = remote_copy(x_ref.at[top], cw0.at[top], 0, right)
        cw_b = remote_copy(x_ref.at[bot], cw0.at[bot], 1, right)
        ccw_a = remote_copy(x_ref.at[bot], ccw0.at[bot], 2, left)
        ccw_b = remote_copy(x_ref.at[top], ccw0.at[top], 3, left)
        cw_a.start()
        cw_b.start()
        ccw_a.start()
        ccw_b.start()

        out_ref[pl.ds(my_pos * m_per, m_per), :] = jnp.dot(
            x_ref[:, :], w_ref[:, :], preferred_element_type=jnp.float32
        )

        cw_a.wait_recv()
        f_cw = remote_copy(cw0.at[top], cw1, 4, right)
        f_cw.start()
        ccw_a.wait_recv()
        f_ccw = remote_copy(ccw0.at[bot], ccw1, 5, left)
        f_ccw.start()

        cw_b.wait_recv()
        ccw_b.wait_recv()
        out_ref[pl.ds(left * m_per, m_per), :] = jnp.dot(
            cw0[:, :], w_ref[:, :], preferred_element_type=jnp.float32
        )
        out_ref[pl.ds(right * m_per, m_per), :] = jnp.dot(
            ccw0[:, :], w_ref[:, :], preferred_element_type=jnp.float32
        )

        diag = (my_pos + 2) % N_DEV
        f_cw.wait_recv()
        out_ref[pl.ds(diag * m_per, half), :] = jnp.dot(
            cw1[:, :], w_ref[:, :], preferred_element_type=jnp.float32
        )
        f_ccw.wait_recv()
        out_ref[pl.ds(diag * m_per + half, half), :] = jnp.dot(
            ccw1[:, :], w_ref[:, :], preferred_element_type=jnp.float32
        )

        for r in (cw_a, cw_b, ccw_a, ccw_b, f_cw, f_ccw):
            r.wait_send()

    out_shape = jax.ShapeDtypeStruct((N_DEV * m_per, n_shard), jnp.float32)
    return pl.pallas_call(
        body,
        out_shape=out_shape,
        in_specs=[
            pl.BlockSpec(memory_space=pltpu.VMEM),
            pl.BlockSpec(memory_space=pltpu.VMEM),
        ],
        out_specs=pl.BlockSpec(memory_space=pltpu.VMEM),
        scratch_shapes=[
            pltpu.VMEM((m_per, k), jnp.float32),
            pltpu.VMEM((m_per, k), jnp.float32),
            pltpu.VMEM((half, k), jnp.float32),
            pltpu.VMEM((half, k), jnp.float32),
            pltpu.SemaphoreType.DMA((6,)),
            pltpu.SemaphoreType.DMA((6,)),
        ],
        compiler_params=pltpu.CompilerParams(collective_id=0),
    )(x, w_mat)
